# baseline (speedup 1.0000x reference)
import jax
import jax.numpy as jnp
from jax.experimental import pallas as pl
from jax.experimental.pallas import tpu as pltpu

_BN = 8


def _wsum_flat_kernel(hw: int, c_in: int, x_ref, w_ref, o_ref):
    acc = w_ref[0, 0] * x_ref[:, 0:hw]
    for c in range(1, c_in):
        acc = acc + w_ref[c, 0] * x_ref[:, c * hw:(c + 1) * hw]
    o_ref[...] = acc


def kernel(x_nchw, weight):
    N, C_in, H, W = x_nchw.shape
    C_out = weight.shape[0]
    HW = H * W
    w_col = weight.reshape(C_out * C_in, 1).astype(jnp.float32)

    bn = _BN
    while N % bn:
        bn //= 2

    x = x_nchw.reshape(N, C_in * HW)
    in_bytes = bn * C_in * HW * x.dtype.itemsize
    vmem = int(min(2 * in_bytes + 2 * bn * HW * 4 + (1 << 20), 100 << 20))

    import functools
    out = pl.pallas_call(
        functools.partial(_wsum_flat_kernel, HW, C_in),
        out_shape=jax.ShapeDtypeStruct((N, HW), x_nchw.dtype),
        grid=(N // bn,),
        in_specs=[
            pl.BlockSpec((bn, C_in * HW), lambda i: (i, 0)),
            pl.BlockSpec(memory_space=pltpu.MemorySpace.SMEM),
        ],
        out_specs=pl.BlockSpec((bn, HW), lambda i: (i, 0)),
        compiler_params=pltpu.CompilerParams(
            dimension_semantics=("arbitrary",),
            vmem_limit_bytes=vmem,
        ),
    )(x, w_col)
    return out.reshape(N, C_out, H, W)
